# baseline (speedup 1.0000x reference)
import functools

import jax
import jax.numpy as jnp
import numpy as np
from jax import lax
from jax.experimental import pallas as pl
from jax.experimental.pallas import tpu as pltpu
from jax.experimental.pallas import tpu_sc as plsc

B = 32
V = 10000
J = 52

NUM_WORKERS = 32
V_PAD = 10240
V_TC = 4096
V_SC = V_PAD - V_TC
VL = V_SC // NUM_WORKERS
GROUPS = VL // 16
_TSTRIDE = 521


def _row16(lanes_vals):
    r = np.zeros(16, np.float32)
    for l, v in lanes_vals:
        r[l] = v
    return r


_ROWS_NP = np.stack([
    _row16([(6, -1.0), (9, 1.0)]),
    _row16([(2, 1.0), (8, -1.0)]),
    _row16([(1, -1.0), (4, 1.0)]),
    _row16([(0, 1), (4, 1), (8, 1)]),
    _row16([(1, 1), (5, 1), (9, 1)]),
    _row16([(2, 1), (6, 1), (10, 1)]),
    _row16([(0, 1), (5, 1), (10, 1)]),
    _row16([(15, 1)]),
    _row16([(3, 1)]),
    _row16([(7, 1)]),
    _row16([(11, 1)]),
    _row16([(3, 1), (7, 1), (11, 1)]),
    _row16([(0, 1)]),
    _row16([(5, 1)]),
    _row16([(10, 1)]),
    _row16([(1, 1), (4, 1)]),
    _row16([(2, 1), (8, 1)]),
    _row16([(6, 1), (9, 1)]),
])


def _perm_mats():
    mats = np.zeros((8, 16, 16), np.float32)
    for k in range(4):
        for r in range(4):
            for c in range(4):
                mats[k, 4 * r + k, 4 * r + c] = 1
                mats[4 + k, 4 * k + c, 4 * r + c] = 1
    return mats


_MATS_NP = _perm_mats()


def _chain_body(poses3_ref, jt_ref, rows_ref, mats_ref, rel_ref, world_ref):
    CKX, CKY, CKZ = rows_ref[0:1, :], rows_ref[1:2, :], rows_ref[2:3, :]
    CTX, CTY, CTZ = rows_ref[3:4, :], rows_ref[4:5, :], rows_ref[5:6, :]
    I3, E15 = rows_ref[6:7, :], rows_ref[7:8, :]
    TRX, TRY, TRZ = rows_ref[8:9, :], rows_ref[9:10, :], rows_ref[10:11, :]
    TMASK = rows_ref[11:12, :]
    D00, D11, D22 = rows_ref[12:13, :], rows_ref[13:14, :], rows_ref[14:15, :]
    OXY, OXZ, OYZ = rows_ref[15:16, :], rows_ref[16:17, :], rows_ref[17:18, :]
    PM = [mats_ref[k] for k in range(4)]
    QM = [mats_ref[4 + k] for k in range(4)]

    def bf(x):
        return x.astype(jnp.bfloat16).astype(jnp.float32)

    def mat4mul(A, Bm):
        out = (A @ PM[0]) * (Bm @ QM[0])
        for k in range(1, 4):
            out = out + (A @ PM[k]) * (Bm @ QM[k])
        return out

    jt = jt_ref[...]

    locals_ = []
    for j in range(J):
        p3 = poses3_ref[j]
        pe = p3 + 1e-8
        a = jnp.sqrt(jnp.sum(pe * pe, axis=1, keepdims=True))
        u = p3 / a
        s, c = jnp.sin(a), jnp.cos(a)
        ux, uy, uz = u[:, 0:1], u[:, 1:2], u[:, 2:3]
        K = ux * CKX + uy * CKY + uz * CKZ
        xb, yb, zb = bf(ux), bf(uy), bf(uz)
        xx, yy, zz = xb * xb, yb * yb, zb * zb
        xy, xz, yz = xb * yb, xb * zb, yb * zb
        K2 = ((-zz - yy) * D00 + (-zz - xx) * D11 + (-yy - xx) * D22
              + xy * OXY + xz * OXZ + yz * OYZ)
        R16 = I3 + s * K + (1 - c) * K2
        jrow = jt[j:j + 1]
        rel = jrow - (jt[j - 1:j] if j > 0 else 0.0)
        trans16 = rel[:, 0:1] * TRX + rel[:, 1:2] * TRY + rel[:, 2:3] * TRZ
        locals_.append(R16 + trans16 + E15)

    def rel_adjust(T, j):
        jrow = jt[j:j + 1]
        jt_tiled = jrow[:, 0:1] * CTX + jrow[:, 1:2] * CTY + jrow[:, 2:3] * CTZ
        tmp = T * jt_tiled
        ssum = (pltpu.roll(tmp, 3, axis=1) + pltpu.roll(tmp, 2, axis=1)
                + pltpu.roll(tmp, 1, axis=1))
        return T - ssum * TMASK

    T = locals_[0]
    rel_ref[0] = rel_adjust(T, 0)
    world_ref[0] = T
    for j in range(1, J):
        T = mat4mul(bf(T), bf(locals_[j]))
        rel_ref[j] = rel_adjust(T, j)
        world_ref[j] = T


def _blend_body(idx_ref, w_ref, vt_ref, rel2_ref, vrows_ref, out_ref):
    iota52 = lax.broadcasted_iota(jnp.int32, (1, J), 1)
    W = 0.0
    for k in range(4):
        onehot = (idx_ref[:, k:k + 1] == iota52).astype(jnp.float32)
        W = W + w_ref[:, k:k + 1] * onehot
    G = rel2_ref[...]
    bfc = lambda x: x.astype(jnp.bfloat16).astype(jnp.float32)
    W1 = bfc(W); W2 = bfc(W - W1)
    G1 = bfc(G); G2 = bfc(G - G1)
    WG = (W1 @ G1 + W1 @ G2) + W2 @ G1
    vh = (vt_ref[:, 0:1] * vrows_ref[0:1, :] + vt_ref[:, 1:2] * vrows_ref[1:2, :]
          + vt_ref[:, 2:3] * vrows_ref[2:3, :] + vrows_ref[3:4, :])
    prod = WG * vh
    s = prod + pltpu.roll(prod, 480, axis=1)
    s = s + pltpu.roll(prod, 448, axis=1)
    s = s + pltpu.roll(prod, 416, axis=1)
    out_ref[...] = jnp.concatenate(
        [s[:, 0:32], s[:, 128:160], s[:, 256:288]], axis=1)


_VROWS_NP = np.zeros((4, 512), np.float32)
for _d in range(4):
    for _c in range(3):
        for _b in range(32):
            _VROWS_NP[_d, (4 * _c + _d) * 32 + _b] = 1.0


def _sc_body(tbl_hbm, idx_hbm, w_hbm, vt_hbm, out_hbm,
             tbl_d, tbl_v, idx_v, w_v, vt_v, out_v, sem):
    wid = lax.axis_index("s") * 2 + lax.axis_index("c")
    vbase0 = V_TC + wid * VL

    copies = [pltpu.make_async_copy(tbl_hbm, tbl_d, sem)]
    for k in range(4):
        copies.append(pltpu.make_async_copy(
            idx_hbm.at[pl.ds(k * V_PAD + vbase0, VL)],
            idx_v.at[pl.ds(k * VL, VL)], sem))
        copies.append(pltpu.make_async_copy(
            w_hbm.at[pl.ds(k * V_PAD + vbase0, VL)],
            w_v.at[pl.ds(k * VL, VL)], sem))
    for d in range(3):
        copies.append(pltpu.make_async_copy(
            vt_hbm.at[pl.ds(d * V_PAD + vbase0, VL)],
            vt_v.at[pl.ds(d * VL, VL)], sem))
    for cp in copies:
        cp.start()
    for cp in copies:
        cp.wait()

    def repack(j, _):
        for t in range(32):
            tbl_v[pl.ds(j * _TSTRIDE + t * 16, 16)] = (
                tbl_d[pl.ds(j * 512 + t * 16, 16)])
        return 0

    lax.fori_loop(0, J, repack, 0)

    def group_body(g, _):
        vb = g * 16
        idxs = [idx_v[pl.ds(k * VL + vb, 16)] for k in range(4)]
        ws = [w_v[pl.ds(k * VL + vb, 16)] for k in range(4)]
        vx = vt_v[pl.ds(0 * VL + vb, 16)]
        vy = vt_v[pl.ds(1 * VL + vb, 16)]
        vz = vt_v[pl.ds(2 * VL + vb, 16)]
        sidx = [ix * _TSTRIDE for ix in idxs]

        def batch_body(b4, _):
            for bu in range(4):
                b = b4 * 4 + bu
                bidx = [sx + b * 16 for sx in sidx]
                m = []
                for p in range(12):
                    acc = ws[0] * plsc.load_gather(tbl_v, [bidx[0] + p])
                    for k in range(1, 4):
                        acc = acc + ws[k] * plsc.load_gather(tbl_v, [bidx[k] + p])
                    m.append(acc)
                ob = (b * 3) * VL + vb
                for c in range(3):
                    o = (m[4 * c] * vx + m[4 * c + 1] * vy
                         + m[4 * c + 2] * vz + m[4 * c + 3])
                    out_v[pl.ds(ob + c * VL, 16)] = o
            return 0

        lax.fori_loop(0, B // 4, batch_body, 0)
        return 0

    lax.fori_loop(0, GROUPS, group_body, 0)

    obase = wid * VL

    def out_start(i, _):
        pltpu.make_async_copy(
            out_v.at[pl.ds(i * VL, VL)],
            out_hbm.at[pl.ds(i * V_SC + obase, VL)], sem).start()
        return 0

    lax.fori_loop(0, B * 3, out_start, 0)

    def out_wait(i, _):
        pltpu.make_async_copy(
            out_v.at[pl.ds(i * VL, VL)],
            out_hbm.at[pl.ds(i * V_SC + obase, VL)], sem).wait()
        return 0

    lax.fori_loop(0, B * 3, out_wait, 0)


@functools.cache
def _sc_blend():
    return pl.kernel(
        _sc_body,
        out_type=jax.ShapeDtypeStruct((B * 3 * V_SC,), jnp.float32),
        mesh=plsc.VectorSubcoreMesh(core_axis_name="c", subcore_axis_name="s"),
        compiler_params=pltpu.CompilerParams(
            needs_layout_passes=False, use_tc_tiling_on_sc=False,
            disable_bounds_checks=True),
        scratch_types=[
            pltpu.VMEM((J * B * 16,), jnp.float32),
            pltpu.VMEM((J * _TSTRIDE,), jnp.float32),
            pltpu.VMEM((4 * VL,), jnp.int32),
            pltpu.VMEM((4 * VL,), jnp.float32),
            pltpu.VMEM((3 * VL,), jnp.float32),
            pltpu.VMEM((B * 3 * VL,), jnp.float32),
            pltpu.SemaphoreType.DMA,
        ],
    )


def kernel(poses, v_template, j_template, skin_weights, skin_indices, parents):
    del parents

    poses3 = poses.reshape(B, J, 3).transpose(1, 0, 2)
    rel, world = pl.pallas_call(
        _chain_body,
        out_shape=[
            jax.ShapeDtypeStruct((J, B, 16), jnp.float32),
            jax.ShapeDtypeStruct((J, B, 16), jnp.float32),
        ],
    )(poses3, j_template, jnp.asarray(_ROWS_NP), jnp.asarray(_MATS_NP))

    posed_joints = world[:, :, 3:12:4].transpose(1, 0, 2)

    idx_t = jnp.zeros((4, V_PAD), jnp.int32).at[:, :V].set(
        skin_indices.astype(jnp.int32).T)
    w_t = jnp.zeros((4, V_PAD), jnp.float32).at[:, :V].set(skin_weights.T)
    vt_t = jnp.zeros((3, V_PAD), jnp.float32).at[:, :V].set(v_template.T)

    out_sc = _sc_blend()(rel.reshape(-1), idx_t.reshape(-1),
                         w_t.reshape(-1), vt_t.reshape(-1))

    out_tc = pl.pallas_call(
        _blend_body,
        out_shape=jax.ShapeDtypeStruct((V_TC, 96), jnp.float32),
    )(skin_indices.astype(jnp.int32)[:V_TC], skin_weights[:V_TC],
      v_template[:V_TC], rel.transpose(0, 2, 1).reshape(J, 16 * B),
      jnp.asarray(_VROWS_NP))

    tc_part = out_tc.reshape(V_TC, 3, 32).transpose(2, 0, 1)
    sc_part = out_sc.reshape(B, 3, V_SC).transpose(0, 2, 1)
    vertices = jnp.concatenate([tc_part, sc_part], axis=1)[:, :V]
    return (vertices, posed_joints)


# speedup vs baseline: 1.1571x; 1.1571x over previous
import functools

import jax
import jax.numpy as jnp
import numpy as np
from jax import lax
from jax.experimental import pallas as pl
from jax.experimental.pallas import tpu as pltpu
from jax.experimental.pallas import tpu_sc as plsc

B = 32
V = 10000
J = 52

NUM_WORKERS = 32
V_PAD = 10240
V_TC = 4096
V_SC = V_PAD - V_TC
VL = V_SC // NUM_WORKERS
GROUPS = VL // 16
_TSTRIDE = 521


def _row16(lanes_vals):
    r = np.zeros(16, np.float32)
    for l, v in lanes_vals:
        r[l] = v
    return r


_ROWS_NP = np.stack([
    _row16([(6, -1.0), (9, 1.0)]),
    _row16([(2, 1.0), (8, -1.0)]),
    _row16([(1, -1.0), (4, 1.0)]),
    _row16([(0, 1), (4, 1), (8, 1)]),
    _row16([(1, 1), (5, 1), (9, 1)]),
    _row16([(2, 1), (6, 1), (10, 1)]),
    _row16([(0, 1), (5, 1), (10, 1)]),
    _row16([(15, 1)]),
    _row16([(3, 1)]),
    _row16([(7, 1)]),
    _row16([(11, 1)]),
    _row16([(3, 1), (7, 1), (11, 1)]),
])


def _perm_mats():
    mats = np.zeros((8, 16, 16), np.float32)
    for k in range(4):
        for r in range(4):
            for c in range(4):
                mats[k, 4 * r + k, 4 * r + c] = 1
                mats[4 + k, 4 * k + c, 4 * r + c] = 1
    return mats


_MATS_NP = _perm_mats()


def _chain_body(poses3_ref, jt_ref, rows_ref, mats_ref, rel_ref, world_ref):
    CKX, CKY, CKZ = rows_ref[0:1, :], rows_ref[1:2, :], rows_ref[2:3, :]
    CTX, CTY, CTZ = rows_ref[3:4, :], rows_ref[4:5, :], rows_ref[5:6, :]
    I3, E15 = rows_ref[6:7, :], rows_ref[7:8, :]
    TRX, TRY, TRZ = rows_ref[8:9, :], rows_ref[9:10, :], rows_ref[10:11, :]
    TMASK = rows_ref[11:12, :]
    PM = [mats_ref[k] for k in range(4)]
    QM = [mats_ref[4 + k] for k in range(4)]

    def bf(x):
        return x.astype(jnp.bfloat16).astype(jnp.float32)

    def mat4mul(A, Bm):
        out = (A @ PM[0]) * (Bm @ QM[0])
        for k in range(1, 4):
            out = out + (A @ PM[k]) * (Bm @ QM[k])
        return out

    jt = jt_ref[...]

    locals_ = []
    for j in range(J):
        p3 = poses3_ref[j]
        pe = p3 + 1e-8
        a = jnp.sqrt(jnp.sum(pe * pe, axis=1, keepdims=True))
        u = p3 / a
        s, c = jnp.sin(a), jnp.cos(a)
        ux, uy, uz = u[:, 0:1], u[:, 1:2], u[:, 2:3]
        K = ux * CKX + uy * CKY + uz * CKZ
        Kb = bf(K)
        K2 = mat4mul(Kb, Kb)
        R16 = I3 + s * K + (1 - c) * K2
        jrow = jt[j:j + 1]
        rel = jrow - (jt[j - 1:j] if j > 0 else 0.0)
        trans16 = rel[:, 0:1] * TRX + rel[:, 1:2] * TRY + rel[:, 2:3] * TRZ
        locals_.append(R16 + trans16 + E15)

    def rel_adjust(T, j):
        jrow = jt[j:j + 1]
        jt_tiled = jrow[:, 0:1] * CTX + jrow[:, 1:2] * CTY + jrow[:, 2:3] * CTZ
        tmp = T * jt_tiled
        ssum = (pltpu.roll(tmp, 3, axis=1) + pltpu.roll(tmp, 2, axis=1)
                + pltpu.roll(tmp, 1, axis=1))
        return T - ssum * TMASK

    T = locals_[0]
    rel_ref[0] = rel_adjust(T, 0)
    world_ref[0] = T
    for j in range(1, J):
        T = mat4mul(bf(T), bf(locals_[j]))
        rel_ref[j] = rel_adjust(T, j)
        world_ref[j] = T


def _blend_body(idx_ref, w_ref, vt_ref, rel2_ref, vrows_ref, out_ref):
    iota52 = lax.broadcasted_iota(jnp.int32, (1, J), 1)
    W = 0.0
    for k in range(4):
        onehot = (idx_ref[:, k:k + 1] == iota52).astype(jnp.float32)
        W = W + w_ref[:, k:k + 1] * onehot
    G = rel2_ref[...]
    bfc = lambda x: x.astype(jnp.bfloat16).astype(jnp.float32)
    W1 = bfc(W); W2 = bfc(W - W1)
    G1 = bfc(G); G2 = bfc(G - G1)
    WG = (W1 @ G1 + W1 @ G2) + W2 @ G1
    vh = (vt_ref[:, 0:1] * vrows_ref[0:1, :] + vt_ref[:, 1:2] * vrows_ref[1:2, :]
          + vt_ref[:, 2:3] * vrows_ref[2:3, :] + vrows_ref[3:4, :])
    prod = WG * vh
    s = prod + pltpu.roll(prod, 480, axis=1)
    s = s + pltpu.roll(prod, 448, axis=1)
    s = s + pltpu.roll(prod, 416, axis=1)
    out_ref[...] = jnp.concatenate(
        [s[:, 0:32], s[:, 128:160], s[:, 256:288]], axis=1)


_VROWS_NP = np.zeros((4, 512), np.float32)
for _d in range(4):
    for _c in range(3):
        for _b in range(32):
            _VROWS_NP[_d, (4 * _c + _d) * 32 + _b] = 1.0


def _sc_body(tbl_hbm, idx_hbm, w_hbm, vt_hbm, out_hbm,
             tbl_d, tbl_v, idx_v, w_v, vt_v, out_v, sem):
    wid = lax.axis_index("s") * 2 + lax.axis_index("c")
    vbase0 = V_TC + wid * VL

    copies = [pltpu.make_async_copy(tbl_hbm, tbl_d, sem)]
    for k in range(4):
        copies.append(pltpu.make_async_copy(
            idx_hbm.at[pl.ds(k * V_PAD + vbase0, VL)],
            idx_v.at[pl.ds(k * VL, VL)], sem))
        copies.append(pltpu.make_async_copy(
            w_hbm.at[pl.ds(k * V_PAD + vbase0, VL)],
            w_v.at[pl.ds(k * VL, VL)], sem))
    for d in range(3):
        copies.append(pltpu.make_async_copy(
            vt_hbm.at[pl.ds(d * V_PAD + vbase0, VL)],
            vt_v.at[pl.ds(d * VL, VL)], sem))
    for cp in copies:
        cp.start()
    for cp in copies:
        cp.wait()

    def repack(j, _):
        for t in range(32):
            tbl_v[pl.ds(j * _TSTRIDE + t * 16, 16)] = (
                tbl_d[pl.ds(j * 512 + t * 16, 16)])
        return 0

    lax.fori_loop(0, J, repack, 0)

    def group_body(g, _):
        vb = g * 16
        idxs = [idx_v[pl.ds(k * VL + vb, 16)] for k in range(4)]
        ws = [w_v[pl.ds(k * VL + vb, 16)] for k in range(4)]
        vx = vt_v[pl.ds(0 * VL + vb, 16)]
        vy = vt_v[pl.ds(1 * VL + vb, 16)]
        vz = vt_v[pl.ds(2 * VL + vb, 16)]
        sidx = [ix * _TSTRIDE for ix in idxs]

        def batch_body(b4, _):
            for bu in range(4):
                b = b4 * 4 + bu
                bidx = [sx + b * 16 for sx in sidx]
                m = []
                for p in range(12):
                    acc = ws[0] * plsc.load_gather(tbl_v, [bidx[0] + p])
                    for k in range(1, 4):
                        acc = acc + ws[k] * plsc.load_gather(tbl_v, [bidx[k] + p])
                    m.append(acc)
                ob = (b * 3) * VL + vb
                for c in range(3):
                    o = (m[4 * c] * vx + m[4 * c + 1] * vy
                         + m[4 * c + 2] * vz + m[4 * c + 3])
                    out_v[pl.ds(ob + c * VL, 16)] = o
            return 0

        lax.fori_loop(0, B // 4, batch_body, 0)
        return 0

    lax.fori_loop(0, GROUPS, group_body, 0)

    obase = wid * VL

    def out_start(i, _):
        pltpu.make_async_copy(
            out_v.at[pl.ds(i * VL, VL)],
            out_hbm.at[pl.ds(i * V_SC + obase, VL)], sem).start()
        return 0

    lax.fori_loop(0, B * 3, out_start, 0)

    def out_wait(i, _):
        pltpu.make_async_copy(
            out_v.at[pl.ds(i * VL, VL)],
            out_hbm.at[pl.ds(i * V_SC + obase, VL)], sem).wait()
        return 0

    lax.fori_loop(0, B * 3, out_wait, 0)


@functools.cache
def _sc_blend():
    return pl.kernel(
        _sc_body,
        out_type=jax.ShapeDtypeStruct((B * 3 * V_SC,), jnp.float32),
        mesh=plsc.VectorSubcoreMesh(core_axis_name="c", subcore_axis_name="s"),
        compiler_params=pltpu.CompilerParams(
            needs_layout_passes=False, use_tc_tiling_on_sc=False,
            disable_bounds_checks=True),
        scratch_types=[
            pltpu.VMEM((J * B * 16,), jnp.float32),
            pltpu.VMEM((J * _TSTRIDE,), jnp.float32),
            pltpu.VMEM((4 * VL,), jnp.int32),
            pltpu.VMEM((4 * VL,), jnp.float32),
            pltpu.VMEM((3 * VL,), jnp.float32),
            pltpu.VMEM((B * 3 * VL,), jnp.float32),
            pltpu.SemaphoreType.DMA,
        ],
    )


def kernel(poses, v_template, j_template, skin_weights, skin_indices, parents):
    del parents

    poses3 = poses.reshape(B, J, 3).transpose(1, 0, 2)
    rel, world = pl.pallas_call(
        _chain_body,
        out_shape=[
            jax.ShapeDtypeStruct((J, B, 16), jnp.float32),
            jax.ShapeDtypeStruct((J, B, 16), jnp.float32),
        ],
    )(poses3, j_template, jnp.asarray(_ROWS_NP), jnp.asarray(_MATS_NP))

    posed_joints = world[:, :, 3:12:4].transpose(1, 0, 2)

    idx_t = jnp.zeros((4, V_PAD), jnp.int32).at[:, :V].set(
        skin_indices.astype(jnp.int32).T)
    w_t = jnp.zeros((4, V_PAD), jnp.float32).at[:, :V].set(skin_weights.T)
    vt_t = jnp.zeros((3, V_PAD), jnp.float32).at[:, :V].set(v_template.T)

    out_sc = _sc_blend()(rel.reshape(-1), idx_t.reshape(-1),
                         w_t.reshape(-1), vt_t.reshape(-1))

    out_tc = pl.pallas_call(
        _blend_body,
        out_shape=jax.ShapeDtypeStruct((V_TC, 96), jnp.float32),
    )(skin_indices.astype(jnp.int32)[:V_TC], skin_weights[:V_TC],
      v_template[:V_TC], rel.transpose(0, 2, 1).reshape(J, 16 * B),
      jnp.asarray(_VROWS_NP))

    tc_part = out_tc.reshape(V_TC, 3, 32).transpose(2, 0, 1)
    sc_part = out_sc.reshape(B, 3, V_SC).transpose(0, 2, 1)
    vertices = jnp.concatenate([tc_part, sc_part], axis=1)[:, :V]
    return (vertices, posed_joints)
